# baseline (speedup 1.0000x reference)
import jax
import jax.numpy as jnp
from jax import lax
from jax.experimental import pallas as pl
from jax.experimental.pallas import tpu as pltpu
from jax.experimental.pallas import tpu_sc as plsc

_DIM = 1024
_SCALE = _DIM ** (-0.5)
_NC = 2
_NS = 16
_NW = _NC * _NS
_F = 8192
_SLOTS = 4


def _sc_body(emb_hbm, out_hbm, *scratch):
    in_bufs = scratch[:_SLOTS]
    out_bufs = scratch[_SLOTS:2 * _SLOTS]
    load_sems = scratch[2 * _SLOTS:3 * _SLOTS]
    store_sems = scratch[3 * _SLOTS:4 * _SLOTS]

    total = emb_hbm.shape[0]
    per_w = total // _NW
    n = per_w // _F

    wid = lax.axis_index("s") * _NC + lax.axis_index("c")
    base = wid * per_w

    def load(i):
        b = i % _SLOTS
        return pltpu.async_copy(
            emb_hbm.at[pl.ds(pl.multiple_of(base + i * _F, 8), _F)],
            in_bufs[b], load_sems[b])

    def store(i):
        b = i % _SLOTS
        return pltpu.async_copy(
            out_bufs[b],
            out_hbm.at[pl.ds(pl.multiple_of(base + i * _F, 8), _F)],
            store_sems[b])

    loads = {i: load(i) for i in range(min(_SLOTS, n))}
    stores = {}
    for i in range(n):
        b = i % _SLOTS
        loads.pop(i).wait()
        if i >= _SLOTS:
            stores.pop(i - _SLOTS).wait()

        ib, ob = in_bufs[b], out_bufs[b]

        @plsc.parallel_loop(0, _F, 16, unroll=8)
        def _scale(j):
            ob[pl.ds(j, 16)] = ib[pl.ds(j, 16)] * _SCALE

        stores[i] = store(i)
        if i + _SLOTS < n:
            loads[i + _SLOTS] = load(i + _SLOTS)
    for i in sorted(stores):
        stores.pop(i).wait()


def kernel(x, emb):
    rows, dim = emb.shape
    total = rows * dim
    assert total % (_NW * _F) == 0
    mesh = plsc.VectorSubcoreMesh(
        core_axis_name="c", subcore_axis_name="s",
        num_cores=_NC, num_subcores=_NS)
    scratch = (
        [pltpu.VMEM((_F,), jnp.float32) for _ in range(2 * _SLOTS)]
        + [pltpu.SemaphoreType.DMA for _ in range(2 * _SLOTS)]
    )
    out_flat = pl.kernel(
        _sc_body,
        out_type=jax.ShapeDtypeStruct((total,), emb.dtype),
        mesh=mesh,
        scratch_types=scratch,
    )(emb.reshape(total))
    return out_flat.reshape(rows, dim)
